# Initial kernel scaffold; baseline (speedup 1.0000x reference)
import jax
import jax.numpy as jnp
from jax.experimental import pallas as pl


def kernel(X, edge_ids, node_ids, W_v2e, b_v2e, W_e2v, b_e2v, e_weight):
    raise NotImplementedError("write your pallas kernel here")


# speedup vs baseline: 3.2254x; 3.2254x over previous
import jax
import jax.numpy as jnp
from jax import lax
from jax.experimental import pallas as pl
from jax.experimental.pallas import tpu as pltpu
from jax.experimental.pallas import tpu_sc as plsc

NV, NE, NNZ, C = 10000, 5000, 320000, 128
NW = 32
PER_TILE = 10240
CHUNK = 128
NCHUNK = PER_TILE // CHUNK
PAD_NNZ = NW * PER_TILE
EACC = 5120
VACC = 10112


def _sc_pass1():
    rows_pt = EACC // 16
    mesh = plsc.VectorSubcoreMesh(core_axis_name="c", subcore_axis_name="s",
                                  num_cores=2, num_subcores=16)

    def body(x_hbm, nid_hbm, eid_hbm, zrow_hbm, zde_hbm, zdv_hbm,
             feat_hbm, degE_hbm, degV_hbm,
             gidx_v, sidx_v, rows_v, histE_v, histV_v, acc_sh, sem):
        c = lax.axis_index("c")
        s = lax.axis_index("s")
        wid = s * 2 + c
        pltpu.sync_copy(zrow_hbm.at[pl.ds(s * rows_pt, rows_pt)],
                        acc_sh.at[pl.ds(s * rows_pt, rows_pt)])
        pltpu.sync_copy(zde_hbm, histE_v)
        pltpu.sync_copy(zdv_hbm, histV_v)
        pltpu.sync_copy(nid_hbm.at[wid], gidx_v)
        pltpu.sync_copy(eid_hbm.at[wid], sidx_v)
        plsc.subcore_barrier()

        ones16 = jnp.ones((16,), jnp.float32)

        def hist(k, carry):
            row = k // 8
            off = (k % 8) * 16
            e16 = sidx_v[row, pl.ds(off, 16)]
            plsc.addupdate_scatter(histE_v, [e16], ones16)
            n16 = gidx_v[row, pl.ds(off, 16)]
            plsc.addupdate_scatter(histV_v, [n16], ones16)
            return carry

        lax.fori_loop(0, NCHUNK * 8, hist, 0)

        def chunk(i, carry):
            pltpu.async_copy(x_hbm.at[gidx_v.at[i]], rows_v, sem).wait()
            pltpu.sync_copy(rows_v, acc_sh.at[sidx_v.at[i]], add=True)
            return carry

        lax.fori_loop(0, NCHUNK, chunk, 0)
        plsc.subcore_barrier()
        pltpu.sync_copy(acc_sh.at[pl.ds(s * rows_pt, rows_pt)],
                        feat_hbm.at[c].at[pl.ds(s * rows_pt, rows_pt)])
        pltpu.sync_copy(histE_v, degE_hbm.at[wid])
        pltpu.sync_copy(histV_v, degV_hbm.at[wid])

    return pl.kernel(
        body,
        out_type=[
            jax.ShapeDtypeStruct((2, EACC, C), jnp.float32),
            jax.ShapeDtypeStruct((NW, EACC), jnp.float32),
            jax.ShapeDtypeStruct((NW, VACC), jnp.float32),
        ],
        mesh=mesh,
        compiler_params=pltpu.CompilerParams(needs_layout_passes=False),
        scratch_types=[
            pltpu.VMEM((NCHUNK, CHUNK), jnp.int32),
            pltpu.VMEM((NCHUNK, CHUNK), jnp.int32),
            pltpu.VMEM((CHUNK, C), jnp.float32),
            pltpu.VMEM((EACC,), jnp.float32),
            pltpu.VMEM((VACC,), jnp.float32),
            pltpu.VMEM_SHARED((EACC, C), jnp.float32),
            pltpu.SemaphoreType.DMA,
        ],
    )


def _sc_pass2():
    rows_pt = VACC // 16
    mesh = plsc.VectorSubcoreMesh(core_axis_name="c", subcore_axis_name="s",
                                  num_cores=2, num_subcores=16)

    def body(y_hbm, eid_hbm, nid_hbm, zrow_hbm, feat_hbm,
             gidx_v, sidx_v, rows_v, acc_sh, sem):
        c = lax.axis_index("c")
        s = lax.axis_index("s")
        wid = s * 2 + c
        pltpu.sync_copy(zrow_hbm.at[pl.ds(s * rows_pt, rows_pt)],
                        acc_sh.at[pl.ds(s * rows_pt, rows_pt)])
        pltpu.sync_copy(eid_hbm.at[wid], gidx_v)
        pltpu.sync_copy(nid_hbm.at[wid], sidx_v)
        plsc.subcore_barrier()

        def chunk(i, carry):
            pltpu.async_copy(y_hbm.at[gidx_v.at[i]], rows_v, sem).wait()
            pltpu.sync_copy(rows_v, acc_sh.at[sidx_v.at[i]], add=True)
            return carry

        lax.fori_loop(0, NCHUNK, chunk, 0)
        plsc.subcore_barrier()
        pltpu.sync_copy(acc_sh.at[pl.ds(s * rows_pt, rows_pt)],
                        feat_hbm.at[c].at[pl.ds(s * rows_pt, rows_pt)])

    return pl.kernel(
        body,
        out_type=jax.ShapeDtypeStruct((2, VACC, C), jnp.float32),
        mesh=mesh,
        scratch_types=[
            pltpu.VMEM((NCHUNK, CHUNK), jnp.int32),
            pltpu.VMEM((NCHUNK, CHUNK), jnp.int32),
            pltpu.VMEM((CHUNK, C), jnp.float32),
            pltpu.VMEM_SHARED((VACC, C), jnp.float32),
            pltpu.SemaphoreType.DMA,
        ],
    )


def _tc_edge(feat_ref, degT_ref, ew_ref, w_ref, b_ref, out_ref):
    a = feat_ref[0] + feat_ref[1]
    d = jnp.sum(degT_ref[...], axis=1, keepdims=True)
    mask = d > 0.0
    inv = jnp.where(mask, 1.0 / jnp.where(mask, d, 1.0), 0.0)
    mean_pre = a * inv
    mean = jnp.dot(mean_pre, w_ref[...], preferred_element_type=jnp.float32)
    mean = jnp.where(mask, mean + b_ref[...], 0.0)
    out_ref[...] = jnp.maximum(ew_ref[...] * mean, 0.0)


def _tc_node(feat_ref, degT_ref, w_ref, b_ref, out_ref):
    a = feat_ref[0] + feat_ref[1]
    d = jnp.sum(degT_ref[...], axis=1, keepdims=True)
    mask = d > 0.0
    inv = jnp.where(mask, 1.0 / jnp.where(mask, d, 1.0), 0.0)
    mean_pre = a * inv
    out = jnp.dot(mean_pre, w_ref[...], preferred_element_type=jnp.float32)
    out = jnp.where(mask, out + b_ref[...], 0.0)
    out_ref[...] = jnp.maximum(out, 0.0)


def kernel(X, edge_ids, node_ids, W_v2e, b_v2e, W_e2v, b_e2v, e_weight):
    f32 = jnp.float32
    i32 = jnp.int32
    pad = PAD_NNZ - NNZ
    shp = (NW, NCHUNK, CHUNK)

    nid_p = jnp.concatenate([node_ids, jnp.full((pad,), NV, i32)]).reshape(shp)
    eid_p = jnp.concatenate([edge_ids, jnp.full((pad,), NE, i32)]).reshape(shp)
    X_pad = jnp.concatenate([X, jnp.zeros((VACC - NV, C), f32)], axis=0)

    featE, degE, degV = _sc_pass1()(
        X_pad, nid_p, eid_p,
        jnp.zeros((EACC, C), f32), jnp.zeros((EACC,), f32),
        jnp.zeros((VACC,), f32))

    ew_pad = jnp.concatenate([e_weight, jnp.zeros((EACC - NE,), f32)])
    Y = pl.pallas_call(
        _tc_edge,
        out_shape=jax.ShapeDtypeStruct((EACC, C), f32),
    )(featE, jnp.transpose(degE), ew_pad.reshape(EACC, 1), W_v2e,
      b_v2e.reshape(1, C))

    featV = _sc_pass2()(Y, eid_p, nid_p, jnp.zeros((VACC, C), f32))
    X_out = pl.pallas_call(
        _tc_node,
        out_shape=jax.ShapeDtypeStruct((VACC, C), f32),
    )(featV, jnp.transpose(degV), W_e2v, b_e2v.reshape(1, C))
    return X_out[:NV]


# speedup vs baseline: 3.7024x; 1.1479x over previous
import functools

import jax
import jax.numpy as jnp
from jax import lax
from jax.experimental import pallas as pl
from jax.experimental.pallas import tpu as pltpu
from jax.experimental.pallas import tpu_sc as plsc

NV, NE, NNZ, C = 10000, 5000, 320000, 128
NW = 32
PER_TILE = 10240
CHUNK = 128
NCHUNK = PER_TILE // CHUNK
PAD_NNZ = NW * PER_TILE
NBUF = 2
NIDX = 2 * NBUF
VACC = 10112
ROWS_PT = VACC // 16


@functools.cache
def _sc_agg():
    mesh = plsc.VectorSubcoreMesh(core_axis_name="c", subcore_axis_name="s",
                                  num_cores=2, num_subcores=16)

    def body(*refs):
        gsems = refs[-NBUF - NIDX:-NIDX]
        isems = refs[-NIDX:]
        (x_hbm, gidx_hbm, sidx_hbm, zrow_hbm, zhist_hbm,
         feat_hbm, deg_hbm,
         gring, sring, rows_v, hist_v, acc_sh) = refs[:-NBUF - NIDX]
        c = lax.axis_index("c")
        s = lax.axis_index("s")
        wid = s * 2 + c
        pltpu.sync_copy(zhist_hbm, hist_v)
        pltpu.sync_copy(zrow_hbm.at[pl.ds(s * ROWS_PT, ROWS_PT)],
                        acc_sh.at[pl.ds(s * ROWS_PT, ROWS_PT)])
        plsc.subcore_barrier()

        ones16 = jnp.ones((16,), jnp.float32)

        def start_idx(i, j):
            pltpu.async_copy(gidx_hbm.at[wid].at[i], gring.at[j], isems[j])
            pltpu.async_copy(sidx_hbm.at[wid].at[i], sring.at[j], isems[j])

        def wait_idx(i, j):
            pltpu.make_async_copy(gidx_hbm.at[wid].at[i], gring.at[j],
                                  isems[j]).wait()
            pltpu.make_async_copy(sidx_hbm.at[wid].at[i], sring.at[j],
                                  isems[j]).wait()

        def start_gather(j, b):
            pltpu.async_copy(x_hbm.at[gring.at[j]], rows_v.at[b], gsems[b])

        def wait_gather(j, b):
            pltpu.make_async_copy(x_hbm.at[gring.at[j]], rows_v.at[b],
                                  gsems[b]).wait()

        for j in range(NIDX):
            start_idx(j, j)
        for b in range(NBUF):
            wait_idx(b, b)
            start_gather(b, b)

        def group(g, carry):
            for j in range(NIDX):
                i = g * NIDX + j
                b = j % NBUF
                for jj in range(CHUNK // 16):
                    s16 = sring[j, pl.ds(jj * 16, 16)]
                    plsc.addupdate_scatter(hist_v, [s16], ones16)
                wait_gather(j, b)
                pltpu.sync_copy(rows_v.at[b], acc_sh.at[sring.at[j]],
                                add=True)

                @pl.when(i + NIDX < NCHUNK)
                def _():
                    start_idx(i + NIDX, j)

                @pl.when(i + NBUF < NCHUNK)
                def _():
                    jn = (j + NBUF) % NIDX
                    wait_idx(i + NBUF, jn)
                    start_gather(jn, b)
            return carry

        lax.fori_loop(0, NCHUNK // NIDX, group, 0)
        plsc.subcore_barrier()
        pltpu.sync_copy(acc_sh.at[pl.ds(s * ROWS_PT, ROWS_PT)],
                        feat_hbm.at[c].at[pl.ds(s * ROWS_PT, ROWS_PT)])
        pltpu.sync_copy(hist_v, deg_hbm.at[wid])

    return pl.kernel(
        body,
        out_type=[
            jax.ShapeDtypeStruct((2, VACC, C), jnp.float32),
            jax.ShapeDtypeStruct((NW, VACC), jnp.float32),
        ],
        mesh=mesh,
        compiler_params=pltpu.CompilerParams(needs_layout_passes=False),
        scratch_types=[
            pltpu.VMEM((NIDX, CHUNK), jnp.int32),
            pltpu.VMEM((NIDX, CHUNK), jnp.int32),
            pltpu.VMEM((NBUF, CHUNK, C), jnp.float32),
            pltpu.VMEM((VACC,), jnp.float32),
            pltpu.VMEM_SHARED((VACC, C), jnp.float32),
        ] + [pltpu.SemaphoreType.DMA] * (NBUF + NIDX),
    )


def _tc_update(feat_ref, degT_ref, wcol_ref, w_ref, b_ref, out_ref):
    a = feat_ref[0] + feat_ref[1]
    d = jnp.sum(degT_ref[...], axis=1, keepdims=True)
    mask = d > 0.0
    inv = jnp.where(mask, 1.0 / jnp.where(mask, d, 1.0), 0.0)
    mean_pre = a * inv
    out = jnp.dot(mean_pre, w_ref[...], preferred_element_type=jnp.float32)
    out = jnp.where(mask, out + b_ref[...], 0.0)
    out_ref[...] = jnp.maximum(wcol_ref[...] * out, 0.0)


def kernel(X, edge_ids, node_ids, W_v2e, b_v2e, W_e2v, b_e2v, e_weight):
    f32 = jnp.float32
    i32 = jnp.int32
    pad = PAD_NNZ - NNZ
    shp = (NW, NCHUNK, CHUNK)

    nid_p = jnp.concatenate([node_ids, jnp.full((pad,), NV, i32)]).reshape(shp)
    eid_p = jnp.concatenate([edge_ids, jnp.full((pad,), NE, i32)]).reshape(shp)
    X_pad = jnp.concatenate([X, jnp.zeros((VACC - NV, C), f32)], axis=0)

    zrow = jnp.zeros((VACC, C), f32)
    zhist = jnp.zeros((VACC,), f32)
    agg = _sc_agg()
    tc = pl.pallas_call(
        _tc_update, out_shape=jax.ShapeDtypeStruct((VACC, C), f32))

    ew_col = jnp.concatenate(
        [e_weight, jnp.zeros((VACC - NE,), f32)]).reshape(VACC, 1)

    def step(x, per):
        gidx, sidx, wcol, w, b = per
        feat, deg = agg(x, gidx, sidx, zrow, zhist)
        return tc(feat, jnp.transpose(deg), wcol, w, b), ()

    xs = (jnp.stack([nid_p, eid_p]),
          jnp.stack([eid_p, nid_p]),
          jnp.stack([ew_col, jnp.ones((VACC, 1), f32)]),
          jnp.stack([W_v2e, W_e2v]),
          jnp.stack([b_v2e.reshape(1, C), b_e2v.reshape(1, C)]))
    x_final, _ = lax.scan(step, X_pad, xs)
    return x_final[:NV]
